# baseline (speedup 1.0000x reference)
import dataclasses
import functools

import jax
import jax.numpy as jnp
from jax import lax
from jax.experimental import pallas as pl
from jax.experimental.pallas import tpu as pltpu
from jax.experimental.pallas import tpu_sc as plsc

_MARGIN = 0.5
_INT_MIN = -2147483648


def _normalize_body(e_ref, o_ref):
    e = e_ref[...]
    ss = jnp.sum(e * e, axis=1, keepdims=True)
    inv = 1.0 / jnp.maximum(jnp.sqrt(ss), 1e-8)
    o_ref[...] = e * inv


def _normalize(embeddings):
    return pl.pallas_call(
        _normalize_body,
        out_shape=jax.ShapeDtypeStruct(embeddings.shape, jnp.float32),
    )(embeddings)


_NC = 2
_NS = 16
_NW = _NC * _NS
_CHUNK = 128


def _sc_sims(table, idx1, idx2):
    P = idx1.shape[0]
    D = table.shape[1]
    per_w = P // _NW
    n_chunks = pl.cdiv(per_w, _CHUNK)
    last_base = per_w - _CHUNK

    mesh = plsc.VectorSubcoreMesh(
        core_axis_name="c", subcore_axis_name="s",
        num_cores=_NC, num_subcores=_NS)

    cp = pltpu.CompilerParams()
    if "needs_layout_passes" in pltpu.CompilerParams.__dataclass_fields__:
        cp = dataclasses.replace(cp, needs_layout_passes=False)

    @functools.partial(
        pl.kernel,
        compiler_params=cp,
        out_type=jax.ShapeDtypeStruct((P,), jnp.float32),
        mesh=mesh,
        scratch_types=[
            pltpu.VMEM((_CHUNK,), jnp.int32),
            pltpu.VMEM((_CHUNK,), jnp.int32),
            pltpu.VMEM((_CHUNK, D), jnp.float32),
            pltpu.VMEM((_CHUNK, D), jnp.float32),
            pltpu.VMEM((_CHUNK,), jnp.float32),
            pltpu.SemaphoreType.DMA,
            pltpu.SemaphoreType.DMA,
        ],
    )
    def sim_kernel(tab_hbm, i1_hbm, i2_hbm, out_hbm,
                   i1_v, i2_v, r1_v, r2_v, s_v, sem1, sem2):
        wid = lax.axis_index("s") * _NC + lax.axis_index("c")
        base_w = wid * per_w
        iota16 = lax.iota(jnp.int32, 16)

        @pl.loop(0, n_chunks)
        def _chunk(j):
            base = base_w + jnp.minimum(j * _CHUNK, last_base)
            pltpu.sync_copy(i1_hbm.at[pl.ds(base, _CHUNK)], i1_v)
            pltpu.sync_copy(i2_hbm.at[pl.ds(base, _CHUNK)], i2_v)
            cp1 = pltpu.async_copy(tab_hbm.at[i1_v], r1_v, sem1)
            cp2 = pltpu.async_copy(tab_hbm.at[i2_v], r2_v, sem2)
            cp1.wait()
            cp2.wait()

            @pl.loop(0, _CHUNK // 16)
            def _group(g):
                pvec = g * 16 + iota16

                def dstep(t, acc):
                    for dd in range(16):
                        dvec = jnp.full((16,), t * 16 + dd, jnp.int32)
                        a = plsc.load_gather(r1_v, [pvec, dvec])
                        b = plsc.load_gather(r2_v, [pvec, dvec])
                        acc = acc + a * b
                    return acc

                acc = lax.fori_loop(
                    0, D // 16, dstep, jnp.zeros((16,), jnp.float32))
                s_v[pl.ds(g * 16, 16)] = acc

            pltpu.sync_copy(s_v, out_hbm.at[pl.ds(base, _CHUNK)])

    return sim_kernel(table, idx1, idx2)


def _loss_body(sim_ref, lab_ref, lt_ref, out_ref):
    sim = sim_ref[...]
    lab = lab_ref[...]
    inv_t = jnp.exp(-lt_ref[0, 0])

    neg = lab == 0
    pos = lab == 1
    n_neg = jnp.sum(neg.astype(jnp.int32))
    n_pos = jnp.sum(pos.astype(jnp.int32))
    k = jnp.minimum(n_neg // 2 + 1, n_neg)

    b = lax.bitcast_convert_type(sim, jnp.int32)
    s = jnp.where(b >= 0, b, _INT_MIN - b)
    s = jnp.where(neg, s, _INT_MIN)

    def bit_step(i, u):
        u_c = u | (jnp.int32(1) << (31 - i))
        t_c = _INT_MIN + u_c
        cnt = jnp.sum((s >= t_c).astype(jnp.int32))
        return jnp.where(cnt >= k, u_c, u)

    u = lax.fori_loop(0, 32, bit_step, jnp.int32(0))
    t_s = _INT_MIN + u

    cnt_gt = jnp.sum((s > t_s).astype(jnp.int32))
    ties = (k - cnt_gt).astype(jnp.float32)
    t_bits = jnp.where(t_s >= 0, t_s, _INT_MIN - t_s)
    t_val = lax.bitcast_convert_type(t_bits, jnp.float32)

    exp_all = jnp.exp(sim * inv_t)
    gt = s > t_s
    exp_t = jnp.exp(t_val * inv_t)
    neg_exp_sum = (jnp.sum(jnp.where(gt, exp_all, 0.0)) + ties * exp_t)
    thresh = jnp.exp(jnp.float32(-_MARGIN) * inv_t)
    hinge_sum = (jnp.sum(jnp.where(gt, jnp.maximum(exp_all - thresh, 0.0), 0.0))
                 + ties * jnp.maximum(exp_t - thresh, 0.0))
    has_k = k > 0
    neg_exp_sum = jnp.where(has_k, neg_exp_sum, 0.0)
    hinge_sum = jnp.where(has_k, hinge_sum, 0.0)

    pos_terms = jnp.log(exp_all + neg_exp_sum) - sim * inv_t
    pos_sum = jnp.sum(jnp.where(pos, pos_terms, 0.0))
    pos_loss = jnp.where(
        n_pos > 0, pos_sum / jnp.maximum(n_pos, 1).astype(jnp.float32), 0.0)
    neg_loss = jnp.where(
        has_k, hinge_sum / jnp.maximum(k, 1).astype(jnp.float32), 0.0)
    out_ref[...] = jnp.broadcast_to(pos_loss + 0.5 * neg_loss, (1, 1))


def _loss(sim2d, lab2d, lt2d, interpret=False):
    return pl.pallas_call(
        _loss_body,
        out_shape=jax.ShapeDtypeStruct((1, 1), jnp.float32),
        interpret=interpret,
    )(sim2d, lab2d, lt2d)


def kernel(embeddings, pairs, log_temp):
    P = pairs.shape[0]
    enorm = _normalize(embeddings)
    idx1 = pairs[:, 0]
    idx2 = pairs[:, 1]
    sims = _sc_sims(enorm, idx1, idx2)
    sim2d = sims.reshape(P // 128, 128)
    lab2d = pairs[:, 2].reshape(P // 128, 128)
    lt2d = log_temp.reshape(1, 1)
    loss = _loss(sim2d, lab2d, lt2d)
    return loss[0, 0]


# speedup vs baseline: 4.3613x; 4.3613x over previous
import dataclasses
import functools

import jax
import jax.numpy as jnp
from jax import lax
from jax.experimental import pallas as pl
from jax.experimental.pallas import tpu as pltpu
from jax.experimental.pallas import tpu_sc as plsc

_MARGIN = 0.5
_INT_MIN = -2147483648


def _normalize_body(e_ref, o_ref):
    e = e_ref[...]
    ss = jnp.sum(e * e, axis=1, keepdims=True)
    inv = 1.0 / jnp.maximum(jnp.sqrt(ss), 1e-8)
    o_ref[...] = e * inv


def _normalize(embeddings):
    return pl.pallas_call(
        _normalize_body,
        out_shape=jax.ShapeDtypeStruct(embeddings.shape, jnp.float32),
    )(embeddings)


_NC = 2
_NS = 16
_NW = _NC * _NS
_CHUNK = 128


def _sc_sims(table, idx1, idx2):
    P = idx1.shape[0]
    D = table.shape[1]
    per_w = P // _NW
    n_chunks = pl.cdiv(per_w, _CHUNK)
    last_base = per_w - _CHUNK

    mesh = plsc.VectorSubcoreMesh(
        core_axis_name="c", subcore_axis_name="s",
        num_cores=_NC, num_subcores=_NS)

    cp = pltpu.CompilerParams()
    if "needs_layout_passes" in pltpu.CompilerParams.__dataclass_fields__:
        cp = dataclasses.replace(cp, needs_layout_passes=False)

    @functools.partial(
        pl.kernel,
        compiler_params=cp,
        out_type=jax.ShapeDtypeStruct((P,), jnp.float32),
        mesh=mesh,
        scratch_types=[
            pltpu.VMEM((per_w,), jnp.int32),
            pltpu.VMEM((per_w,), jnp.int32),
            pltpu.VMEM((_CHUNK, D), jnp.float32),
            pltpu.VMEM((_CHUNK, D), jnp.float32),
            pltpu.VMEM((16, 17), jnp.float32),
            pltpu.VMEM((_CHUNK,), jnp.float32),
            pltpu.SemaphoreType.DMA,
            pltpu.SemaphoreType.DMA,
        ],
    )
    def sim_kernel(tab_hbm, i1_hbm, i2_hbm, out_hbm,
                   i1_v, i2_v, r1_v, r2_v, t_v, s_v, sem1, sem2):
        wid = lax.axis_index("s") * _NC + lax.axis_index("c")
        base_w = wid * per_w
        iota16 = lax.iota(jnp.int32, 16)
        pltpu.sync_copy(i1_hbm.at[pl.ds(base_w, per_w)], i1_v)
        pltpu.sync_copy(i2_hbm.at[pl.ds(base_w, per_w)], i2_v)

        @pl.loop(0, n_chunks)
        def _chunk(j):
            off = jnp.minimum(j * _CHUNK, last_base)
            base = base_w + off
            cp1 = pltpu.async_copy(
                tab_hbm.at[i1_v.at[pl.ds(off, _CHUNK)]], r1_v, sem1)
            cp2 = pltpu.async_copy(
                tab_hbm.at[i2_v.at[pl.ds(off, _CHUNK)]], r2_v, sem2)
            cp1.wait()
            cp2.wait()

            @pl.loop(0, _CHUNK // 16)
            def _group(g):
                for i in range(16):
                    p = g * 16 + i
                    acc = (r1_v[p, pl.ds(0, 16)] * r2_v[p, pl.ds(0, 16)])
                    for c in range(1, D // 16):
                        acc = acc + (r1_v[p, pl.ds(c * 16, 16)]
                                     * r2_v[p, pl.ds(c * 16, 16)])
                    t_v[i, pl.ds(0, 16)] = acc
                rs = plsc.load_gather(
                    t_v, [iota16, jnp.full((16,), 0, jnp.int32)])
                for l in range(1, 16):
                    rs = rs + plsc.load_gather(
                        t_v, [iota16, jnp.full((16,), l, jnp.int32)])
                s_v[pl.ds(g * 16, 16)] = rs

            pltpu.sync_copy(s_v, out_hbm.at[pl.ds(base, _CHUNK)])

    return sim_kernel(table, idx1, idx2)


def _loss_body(sim_ref, lab_ref, lt_ref, out_ref):
    sim = sim_ref[...]
    lab = lab_ref[...]
    inv_t = jnp.exp(-lt_ref[0, 0])

    neg = lab == 0
    pos = lab == 1
    n_neg = jnp.sum(neg.astype(jnp.int32))
    n_pos = jnp.sum(pos.astype(jnp.int32))
    k = jnp.minimum(n_neg // 2 + 1, n_neg)

    b = lax.bitcast_convert_type(sim, jnp.int32)
    s = jnp.where(b >= 0, b, _INT_MIN - b)
    s = jnp.where(neg, s, _INT_MIN)

    def bit_step(i, u):
        u_c = u | (jnp.int32(1) << (31 - i))
        t_c = _INT_MIN + u_c
        cnt = jnp.sum((s >= t_c).astype(jnp.int32))
        return jnp.where(cnt >= k, u_c, u)

    u = lax.fori_loop(0, 32, bit_step, jnp.int32(0))
    t_s = _INT_MIN + u

    cnt_gt = jnp.sum((s > t_s).astype(jnp.int32))
    ties = (k - cnt_gt).astype(jnp.float32)
    t_bits = jnp.where(t_s >= 0, t_s, _INT_MIN - t_s)
    t_val = lax.bitcast_convert_type(t_bits, jnp.float32)

    exp_all = jnp.exp(sim * inv_t)
    gt = s > t_s
    exp_t = jnp.exp(t_val * inv_t)
    neg_exp_sum = (jnp.sum(jnp.where(gt, exp_all, 0.0)) + ties * exp_t)
    thresh = jnp.exp(jnp.float32(-_MARGIN) * inv_t)
    hinge_sum = (jnp.sum(jnp.where(gt, jnp.maximum(exp_all - thresh, 0.0), 0.0))
                 + ties * jnp.maximum(exp_t - thresh, 0.0))
    has_k = k > 0
    neg_exp_sum = jnp.where(has_k, neg_exp_sum, 0.0)
    hinge_sum = jnp.where(has_k, hinge_sum, 0.0)

    pos_terms = jnp.log(exp_all + neg_exp_sum) - sim * inv_t
    pos_sum = jnp.sum(jnp.where(pos, pos_terms, 0.0))
    pos_loss = jnp.where(
        n_pos > 0, pos_sum / jnp.maximum(n_pos, 1).astype(jnp.float32), 0.0)
    neg_loss = jnp.where(
        has_k, hinge_sum / jnp.maximum(k, 1).astype(jnp.float32), 0.0)
    out_ref[...] = jnp.broadcast_to(pos_loss + 0.5 * neg_loss, (1, 1))


def _loss(sim2d, lab2d, lt2d, interpret=False):
    return pl.pallas_call(
        _loss_body,
        out_shape=jax.ShapeDtypeStruct((1, 1), jnp.float32),
        interpret=interpret,
    )(sim2d, lab2d, lt2d)


def kernel(embeddings, pairs, log_temp):
    P = pairs.shape[0]
    enorm = _normalize(embeddings)
    idx1 = pairs[:, 0]
    idx2 = pairs[:, 1]
    sims = _sc_sims(enorm, idx1, idx2)
    sim2d = sims.reshape(P // 128, 128)
    lab2d = pairs[:, 2].reshape(P // 128, 128)
    lt2d = log_temp.reshape(1, 1)
    loss = _loss(sim2d, lab2d, lt2d)
    return loss[0, 0]


# speedup vs baseline: 6.9300x; 1.5890x over previous
import dataclasses
import functools

import jax
import jax.numpy as jnp
from jax import lax
from jax.experimental import pallas as pl
from jax.experimental.pallas import tpu as pltpu
from jax.experimental.pallas import tpu_sc as plsc

_MARGIN = 0.5
_INT_MIN = -2147483648


def _normalize_body(e_ref, o_ref):
    e = e_ref[...]
    ss = jnp.sum(e * e, axis=1, keepdims=True)
    inv = 1.0 / jnp.maximum(jnp.sqrt(ss), 1e-8)
    o_ref[...] = e * inv


def _normalize(embeddings):
    return pl.pallas_call(
        _normalize_body,
        out_shape=jax.ShapeDtypeStruct(embeddings.shape, jnp.float32),
    )(embeddings)


_NC = 2
_NS = 16
_NW = _NC * _NS
_CHUNK = 64


def _sc_sims(table, idx1, idx2):
    P = idx1.shape[0]
    D = table.shape[1]
    per_w = P // _NW
    n_chunks = pl.cdiv(per_w, _CHUNK)
    n_chunks += n_chunks % 2
    last_base = per_w - _CHUNK

    mesh = plsc.VectorSubcoreMesh(
        core_axis_name="c", subcore_axis_name="s",
        num_cores=_NC, num_subcores=_NS)

    cp = pltpu.CompilerParams()
    if "needs_layout_passes" in pltpu.CompilerParams.__dataclass_fields__:
        cp = dataclasses.replace(cp, needs_layout_passes=False)

    @functools.partial(
        pl.kernel,
        compiler_params=cp,
        out_type=jax.ShapeDtypeStruct((P,), jnp.float32),
        mesh=mesh,
        scratch_types=[
            pltpu.VMEM((per_w,), jnp.int32),
            pltpu.VMEM((per_w,), jnp.int32),
            pltpu.VMEM((2, _CHUNK, D), jnp.float32),
            pltpu.VMEM((2, _CHUNK, D), jnp.float32),
            pltpu.VMEM((16, 17), jnp.float32),
            pltpu.VMEM((per_w,), jnp.float32),
            pltpu.SemaphoreType.DMA,
            pltpu.SemaphoreType.DMA,
            pltpu.SemaphoreType.DMA,
            pltpu.SemaphoreType.DMA,
        ],
    )
    def sim_kernel(tab_hbm, i1_hbm, i2_hbm, out_hbm,
                   i1_v, i2_v, r1_v, r2_v, t_v, s_v,
                   sem1a, sem1b, sem2a, sem2b):
        wid = lax.axis_index("s") * _NC + lax.axis_index("c")
        base_w = wid * per_w
        iota16 = lax.iota(jnp.int32, 16)
        sems1 = (sem1a, sem1b)
        sems2 = (sem2a, sem2b)
        pltpu.sync_copy(i1_hbm.at[pl.ds(base_w, per_w)], i1_v)
        pltpu.sync_copy(i2_hbm.at[pl.ds(base_w, per_w)], i2_v)

        def issue(j, b):
            off = jnp.minimum(j * _CHUNK, last_base)
            pltpu.async_copy(
                tab_hbm.at[i1_v.at[pl.ds(off, _CHUNK)]], r1_v.at[b],
                sems1[b])
            pltpu.async_copy(
                tab_hbm.at[i2_v.at[pl.ds(off, _CHUNK)]], r2_v.at[b],
                sems2[b])

        def wait(b):
            pltpu.make_async_copy(
                tab_hbm.at[i1_v.at[pl.ds(0, _CHUNK)]], r1_v.at[b],
                sems1[b]).wait()
            pltpu.make_async_copy(
                tab_hbm.at[i2_v.at[pl.ds(0, _CHUNK)]], r2_v.at[b],
                sems2[b]).wait()

        issue(0, 0)

        @pl.loop(0, n_chunks, step=2)
        def _chunk(j0):
            for b in range(2):
                j = j0 + b

                @pl.when(j + 1 < n_chunks)
                def _():
                    issue(j + 1, 1 - b)

                wait(b)
                off = jnp.minimum(j * _CHUNK, last_base)

                @pl.loop(0, _CHUNK // 16)
                def _group(g, _b=b, _off=off):
                    rv1 = r1_v.at[_b]
                    rv2 = r2_v.at[_b]
                    for i in range(16):
                        p = g * 16 + i
                        acc = rv1[p, pl.ds(0, 16)] * rv2[p, pl.ds(0, 16)]
                        for c in range(1, D // 16):
                            acc = acc + (rv1[p, pl.ds(c * 16, 16)]
                                         * rv2[p, pl.ds(c * 16, 16)])
                        t_v[i, pl.ds(0, 16)] = acc
                    rs = plsc.load_gather(
                        t_v, [iota16, jnp.full((16,), 0, jnp.int32)])
                    for l in range(1, 16):
                        rs = rs + plsc.load_gather(
                            t_v, [iota16, jnp.full((16,), l, jnp.int32)])
                    s_v[pl.ds(_off + g * 16, 16)] = rs

        pltpu.sync_copy(s_v, out_hbm.at[pl.ds(base_w, per_w)])

    return sim_kernel(table, idx1, idx2)


def _loss_body(sim_ref, lab_ref, lt_ref, out_ref):
    sim = sim_ref[...]
    lab = lab_ref[...]
    inv_t = jnp.exp(-lt_ref[0, 0])

    neg = lab == 0
    pos = lab == 1
    n_neg = jnp.sum(neg.astype(jnp.int32))
    n_pos = jnp.sum(pos.astype(jnp.int32))
    k = jnp.minimum(n_neg // 2 + 1, n_neg)

    b = lax.bitcast_convert_type(sim, jnp.int32)
    s = jnp.where(b >= 0, b, _INT_MIN - b)
    s = jnp.where(neg, s, _INT_MIN)

    def bit_step(i, u):
        u_c = u | (jnp.int32(1) << (31 - i))
        t_c = _INT_MIN + u_c
        cnt = jnp.sum((s >= t_c).astype(jnp.int32))
        return jnp.where(cnt >= k, u_c, u)

    u = lax.fori_loop(0, 32, bit_step, jnp.int32(0))
    t_s = _INT_MIN + u

    cnt_gt = jnp.sum((s > t_s).astype(jnp.int32))
    ties = (k - cnt_gt).astype(jnp.float32)
    t_bits = jnp.where(t_s >= 0, t_s, _INT_MIN - t_s)
    t_val = lax.bitcast_convert_type(t_bits, jnp.float32)

    exp_all = jnp.exp(sim * inv_t)
    gt = s > t_s
    exp_t = jnp.exp(t_val * inv_t)
    neg_exp_sum = (jnp.sum(jnp.where(gt, exp_all, 0.0)) + ties * exp_t)
    thresh = jnp.exp(jnp.float32(-_MARGIN) * inv_t)
    hinge_sum = (jnp.sum(jnp.where(gt, jnp.maximum(exp_all - thresh, 0.0), 0.0))
                 + ties * jnp.maximum(exp_t - thresh, 0.0))
    has_k = k > 0
    neg_exp_sum = jnp.where(has_k, neg_exp_sum, 0.0)
    hinge_sum = jnp.where(has_k, hinge_sum, 0.0)

    pos_terms = jnp.log(exp_all + neg_exp_sum) - sim * inv_t
    pos_sum = jnp.sum(jnp.where(pos, pos_terms, 0.0))
    pos_loss = jnp.where(
        n_pos > 0, pos_sum / jnp.maximum(n_pos, 1).astype(jnp.float32), 0.0)
    neg_loss = jnp.where(
        has_k, hinge_sum / jnp.maximum(k, 1).astype(jnp.float32), 0.0)
    out_ref[...] = jnp.broadcast_to(pos_loss + 0.5 * neg_loss, (1, 1))


def _loss(sim2d, lab2d, lt2d, interpret=False):
    return pl.pallas_call(
        _loss_body,
        out_shape=jax.ShapeDtypeStruct((1, 1), jnp.float32),
        interpret=interpret,
    )(sim2d, lab2d, lt2d)


def kernel(embeddings, pairs, log_temp):
    P = pairs.shape[0]
    enorm = _normalize(embeddings)
    idx1 = pairs[:, 0]
    idx2 = pairs[:, 1]
    sims = _sc_sims(enorm, idx1, idx2)
    sim2d = sims.reshape(P // 128, 128)
    lab2d = pairs[:, 2].reshape(P // 128, 128)
    lt2d = log_temp.reshape(1, 1)
    loss = _loss(sim2d, lab2d, lt2d)
    return loss[0, 0]
